# baseline (device time: 168381 ns/iter reference)
import jax
import jax.numpy as jnp
from jax import lax
from jax.experimental import pallas as pl
from jax.experimental.pallas import tpu as pltpu

N_DEV = 4
SQ = 1024
SKV = 1024
HQ = 32
DH = 128
D_MODEL = 1024
H_PER = HQ // N_DEV
SCALE = 0.08838834764831843
WINDOW = 128
NQB = 8
QB = SQ // NQB
KB = QB + 2 * WINDOW
KSTARTS = [min(max(i * QB - WINDOW, 0), SKV - KB) for i in range(NQB)]


def kernel(x, Wq, K_ext, V_ext, Wo):
    xb = x.astype(jnp.bfloat16)
    wqb = (Wq * SCALE).astype(jnp.bfloat16).reshape(
        D_MODEL, H_PER, DH).transpose(1, 0, 2)
    wob = Wo.astype(jnp.bfloat16).reshape(H_PER, DH, D_MODEL)

    def body(x_ref, wq_ref, k_hbm, v_hbm, wo_ref, out_ref,
             comm_q, comm_o, k_vmem, v_vmem, ctx_buf,
             sq_send, sq_recv, so_send, so_recv, k_sems, v_sems):
        my = lax.axis_index("i")
        right = lax.rem(my + 1, N_DEV)
        left = lax.rem(my + N_DEV - 1, N_DEV)

        comm_q[0] = wq_ref[...]
        comm_o[0] = wo_ref[...]

        def start_kv(h, j):
            slot = h % 2
            cps = []
            for hh in range(H_PER):
                cps.append(pltpu.make_async_copy(
                    k_hbm.at[my, :, j * H_PER + hh, :],
                    k_vmem.at[slot, hh], k_sems.at[slot]))
                cps.append(pltpu.make_async_copy(
                    v_hbm.at[my, :, j * H_PER + hh, :],
                    v_vmem.at[slot, hh], v_sems.at[slot]))
            for c in cps:
                c.start()
            return cps

        OFFS = [0, 3, 2, 1]

        kv = [None] * N_DEV
        kv[0] = start_kv(0, my)

        barrier = pltpu.get_barrier_semaphore()
        pl.semaphore_signal(barrier, inc=1, device_id=(left,),
                            device_id_type=pl.DeviceIdType.MESH)
        pl.semaphore_signal(barrier, inc=1, device_id=(right,),
                            device_id_type=pl.DeviceIdType.MESH)
        pl.semaphore_wait(barrier, 2)

        diff = (lax.broadcasted_iota(jnp.int32, (QB, KB), 0)
                - lax.broadcasted_iota(jnp.int32, (QB, KB), 1))
        neg = jnp.float32(-1e9)

        out_ref[0, :, :] = jnp.zeros((SQ, D_MODEL), jnp.float32)

        for h in range(N_DEV):
            if h < N_DEV - 1:
                rdma_q = pltpu.make_async_remote_copy(
                    src_ref=comm_q.at[h], dst_ref=comm_q.at[h + 1],
                    send_sem=sq_send.at[h], recv_sem=sq_recv.at[h],
                    device_id=(right,), device_id_type=pl.DeviceIdType.MESH)
                rdma_o = pltpu.make_async_remote_copy(
                    src_ref=comm_o.at[h], dst_ref=comm_o.at[h + 1],
                    send_sem=so_send.at[h], recv_sem=so_recv.at[h],
                    device_id=(left,), device_id_type=pl.DeviceIdType.MESH)
                rdma_q.start()
                rdma_o.start()

            for c in kv[h]:
                c.wait()
            if h < N_DEV - 1:
                j_next = lax.rem(my + OFFS[h + 1], N_DEV)
                kv[h + 1] = start_kv(h + 1, j_next)

            slot = h % 2
            oq = {0: 0, 1: None, 2: 2, 3: 1}[h]

            def head_body(hh, carry, h=h, slot=slot, oq=oq):
                for blk in range(NQB):
                    qs = blk * QB
                    ks = KSTARTS[blk]
                    q_b = jnp.dot(x_ref[0, qs:qs + QB, :], comm_q[h, hh],
                                  preferred_element_type=jnp.float32)
                    q_b = q_b.astype(jnp.bfloat16)
                    k_b = k_vmem[slot, hh, ks:ks + KB, :].astype(jnp.bfloat16)
                    v_b = v_vmem[slot, hh, ks:ks + KB, :].astype(jnp.bfloat16)
                    s = lax.dot_general(q_b, k_b, (((1,), (1,)), ((), ())),
                                        preferred_element_type=jnp.float32)
                    shift = qs - ks
                    band = ((diff >= -WINDOW - shift)
                            & (diff <= WINDOW - shift))
                    e = jnp.exp(jnp.where(band, s, neg))
                    den = jnp.sum(e, axis=1, keepdims=True)
                    ctx = jnp.dot(e.astype(jnp.bfloat16), v_b,
                                  preferred_element_type=jnp.float32)
                    ctx = (ctx / den).astype(jnp.bfloat16)
                    if oq is None:
                        ctx_buf[hh, qs:qs + QB, :] = ctx
                    else:
                        woh = comm_o[oq, hh]
                        out_ref[0, qs:qs + QB, :] = (
                            out_ref[0, qs:qs + QB, :]
                            + jnp.dot(ctx, woh,
                                      preferred_element_type=jnp.float32))
                return carry

            lax.fori_loop(0, H_PER, head_body, 0)

            if h == N_DEV - 1:
                def proj_body(hh, carry):
                    out_ref[0, :, :] = (
                        out_ref[0, :, :]
                        + jnp.dot(ctx_buf[hh], comm_o[N_DEV - 1, hh],
                                  preferred_element_type=jnp.float32))
                    return carry

                lax.fori_loop(0, H_PER, proj_body, 0)

            if h < N_DEV - 1:
                rdma_q.wait()
                rdma_o.wait()

    return pl.pallas_call(
        body,
        out_shape=jax.ShapeDtypeStruct((1, SQ, D_MODEL), jnp.float32),
        in_specs=[
            pl.BlockSpec(memory_space=pltpu.MemorySpace.VMEM),
            pl.BlockSpec(memory_space=pltpu.MemorySpace.VMEM),
            pl.BlockSpec(memory_space=pltpu.MemorySpace.HBM),
            pl.BlockSpec(memory_space=pltpu.MemorySpace.HBM),
            pl.BlockSpec(memory_space=pltpu.MemorySpace.VMEM),
        ],
        out_specs=pl.BlockSpec(memory_space=pltpu.MemorySpace.VMEM),
        scratch_shapes=[
            pltpu.VMEM((N_DEV, H_PER, D_MODEL, DH), jnp.bfloat16),
            pltpu.VMEM((N_DEV, H_PER, DH, D_MODEL), jnp.bfloat16),
            pltpu.VMEM((2, H_PER, SKV, DH), jnp.float32),
            pltpu.VMEM((2, H_PER, SKV, DH), jnp.float32),
            pltpu.VMEM((H_PER, SQ, DH), jnp.bfloat16),
            pltpu.SemaphoreType.DMA((N_DEV - 1,)),
            pltpu.SemaphoreType.DMA((N_DEV - 1,)),
            pltpu.SemaphoreType.DMA((N_DEV - 1,)),
            pltpu.SemaphoreType.DMA((N_DEV - 1,)),
            pltpu.SemaphoreType.DMA((2,)),
            pltpu.SemaphoreType.DMA((2,)),
        ],
        compiler_params=pltpu.CompilerParams(
            collective_id=0,
            vmem_limit_bytes=60 * 1024 * 1024,
        ),
    )(xb, wqb, K_ext, V_ext, wob)


# device time: 126956 ns/iter; 1.3263x vs baseline; 1.3263x over previous
import jax
import jax.numpy as jnp
from jax import lax
from jax.experimental import pallas as pl
from jax.experimental.pallas import tpu as pltpu

N_DEV = 4
SQ = 1024
SKV = 1024
HQ = 32
DH = 128
D_MODEL = 1024
H_PER = HQ // N_DEV
SCALE = 0.08838834764831843
WINDOW = 128
NQB = 4
QB = SQ // NQB
KB = QB + 2 * WINDOW
KSTARTS = [min(max(i * QB - WINDOW, 0), SKV - KB) for i in range(NQB)]


def kernel(x, Wq, K_ext, V_ext, Wo):
    xb = x.astype(jnp.bfloat16)
    wqb = (Wq * SCALE).astype(jnp.bfloat16).reshape(
        D_MODEL, H_PER, DH).transpose(1, 0, 2)
    wob = Wo.astype(jnp.bfloat16).reshape(H_PER, DH, D_MODEL)

    def body(x_ref, wq_ref, k_hbm, v_hbm, wo_ref, out_ref,
             comm_q, comm_o, k_vmem, v_vmem, ctx_buf,
             sq_send, sq_recv, so_send, so_recv, k_sems, v_sems):
        my = lax.axis_index("i")
        right = lax.rem(my + 1, N_DEV)
        left = lax.rem(my + N_DEV - 1, N_DEV)

        comm_q[0] = wq_ref[...]
        comm_o[0] = wo_ref[...]

        def start_kv(h, j):
            slot = h % 2
            cps = []
            for hh in range(H_PER):
                cps.append(pltpu.make_async_copy(
                    k_hbm.at[my, :, j * H_PER + hh, :],
                    k_vmem.at[slot, hh], k_sems.at[slot]))
                cps.append(pltpu.make_async_copy(
                    v_hbm.at[my, :, j * H_PER + hh, :],
                    v_vmem.at[slot, hh], v_sems.at[slot]))
            for c in cps:
                c.start()
            return cps

        OFFS = [0, 3, 2, 1]

        kv = [None] * N_DEV
        kv[0] = start_kv(0, my)

        barrier = pltpu.get_barrier_semaphore()
        pl.semaphore_signal(barrier, inc=1, device_id=(left,),
                            device_id_type=pl.DeviceIdType.MESH)
        pl.semaphore_signal(barrier, inc=1, device_id=(right,),
                            device_id_type=pl.DeviceIdType.MESH)
        pl.semaphore_wait(barrier, 2)

        diff = (lax.broadcasted_iota(jnp.int32, (QB, KB), 0)
                - lax.broadcasted_iota(jnp.int32, (QB, KB), 1))
        neg = jnp.float32(-1e9)

        out_ref[0, :, :] = jnp.zeros((SQ, D_MODEL), jnp.float32)

        for h in range(N_DEV):
            if h < N_DEV - 1:
                rdma_q = pltpu.make_async_remote_copy(
                    src_ref=comm_q.at[h], dst_ref=comm_q.at[h + 1],
                    send_sem=sq_send.at[h], recv_sem=sq_recv.at[h],
                    device_id=(right,), device_id_type=pl.DeviceIdType.MESH)
                rdma_o = pltpu.make_async_remote_copy(
                    src_ref=comm_o.at[h], dst_ref=comm_o.at[h + 1],
                    send_sem=so_send.at[h], recv_sem=so_recv.at[h],
                    device_id=(left,), device_id_type=pl.DeviceIdType.MESH)
                rdma_q.start()
                rdma_o.start()

            for c in kv[h]:
                c.wait()
            if h < N_DEV - 1:
                j_next = lax.rem(my + OFFS[h + 1], N_DEV)
                kv[h + 1] = start_kv(h + 1, j_next)

            slot = h % 2
            oq = {0: 0, 1: None, 2: 2, 3: 1}[h]

            def head_body(hh, carry, h=h, slot=slot, oq=oq):
                for blk in range(NQB):
                    qs = blk * QB
                    ks = KSTARTS[blk]
                    q_b = jnp.dot(x_ref[0, qs:qs + QB, :], comm_q[h, hh],
                                  preferred_element_type=jnp.float32)
                    q_b = q_b.astype(jnp.bfloat16)
                    k_b = k_vmem[slot, hh, ks:ks + KB, :].astype(jnp.bfloat16)
                    v_b = v_vmem[slot, hh, ks:ks + KB, :].astype(jnp.bfloat16)
                    s = lax.dot_general(q_b, k_b, (((1,), (1,)), ((), ())),
                                        preferred_element_type=jnp.float32)
                    shift = qs - ks
                    band = ((diff >= -WINDOW - shift)
                            & (diff <= WINDOW - shift))
                    e = jnp.exp(jnp.where(band, s, neg))
                    den = jnp.sum(e, axis=1, keepdims=True)
                    ctx = jnp.dot(e.astype(jnp.bfloat16), v_b,
                                  preferred_element_type=jnp.float32)
                    ctx = (ctx / den).astype(jnp.bfloat16)
                    if oq is None:
                        ctx_buf[hh, qs:qs + QB, :] = ctx
                    else:
                        woh = comm_o[oq, hh]
                        out_ref[0, qs:qs + QB, :] = (
                            out_ref[0, qs:qs + QB, :]
                            + jnp.dot(ctx, woh,
                                      preferred_element_type=jnp.float32))
                return carry

            lax.fori_loop(0, H_PER, head_body, 0)

            if h == N_DEV - 1:
                def proj_body(hh, carry):
                    out_ref[0, :, :] = (
                        out_ref[0, :, :]
                        + jnp.dot(ctx_buf[hh], comm_o[N_DEV - 1, hh],
                                  preferred_element_type=jnp.float32))
                    return carry

                lax.fori_loop(0, H_PER, proj_body, 0)

            if h < N_DEV - 1:
                rdma_q.wait()
                rdma_o.wait()

    return pl.pallas_call(
        body,
        out_shape=jax.ShapeDtypeStruct((1, SQ, D_MODEL), jnp.float32),
        in_specs=[
            pl.BlockSpec(memory_space=pltpu.MemorySpace.VMEM),
            pl.BlockSpec(memory_space=pltpu.MemorySpace.VMEM),
            pl.BlockSpec(memory_space=pltpu.MemorySpace.HBM),
            pl.BlockSpec(memory_space=pltpu.MemorySpace.HBM),
            pl.BlockSpec(memory_space=pltpu.MemorySpace.VMEM),
        ],
        out_specs=pl.BlockSpec(memory_space=pltpu.MemorySpace.VMEM),
        scratch_shapes=[
            pltpu.VMEM((N_DEV, H_PER, D_MODEL, DH), jnp.bfloat16),
            pltpu.VMEM((N_DEV, H_PER, DH, D_MODEL), jnp.bfloat16),
            pltpu.VMEM((2, H_PER, SKV, DH), jnp.float32),
            pltpu.VMEM((2, H_PER, SKV, DH), jnp.float32),
            pltpu.VMEM((H_PER, SQ, DH), jnp.bfloat16),
            pltpu.SemaphoreType.DMA((N_DEV - 1,)),
            pltpu.SemaphoreType.DMA((N_DEV - 1,)),
            pltpu.SemaphoreType.DMA((N_DEV - 1,)),
            pltpu.SemaphoreType.DMA((N_DEV - 1,)),
            pltpu.SemaphoreType.DMA((2,)),
            pltpu.SemaphoreType.DMA((2,)),
        ],
        compiler_params=pltpu.CompilerParams(
            collective_id=0,
            vmem_limit_bytes=60 * 1024 * 1024,
        ),
    )(xb, wqb, K_ext, V_ext, wob)
